# baseline (speedup 1.0000x reference)
import functools

import jax
import jax.numpy as jnp
from jax import lax
from jax.experimental import pallas as pl
from jax.experimental.pallas import tpu as pltpu
from jax.experimental.pallas import tpu_sc as plsc

NUM_CLASSES = 1000000
EMBED = 64
BATCH = 4096
SEQ = 200

_NC = 2
_NS = 16
_NW = _NC * _NS
_ROWS = BATCH * SEQ
_G = 128
_NG_TOTAL = _ROWS // _G
_NG_W = _NG_TOTAL // _NW


def _sc_gather(idx2d, table):
  mesh = plsc.VectorSubcoreMesh(core_axis_name="c", subcore_axis_name="s")

  @functools.partial(
      pl.kernel,
      out_type=jax.ShapeDtypeStruct((_ROWS, EMBED), jnp.float32),
      mesh=mesh,
      compiler_params=pltpu.CompilerParams(use_tc_tiling_on_sc=False),
      scratch_types=[
          pltpu.VMEM((_NG_W, _G), jnp.int32),
          pltpu.VMEM((_G, EMBED), jnp.float32),
          pltpu.VMEM((_G, EMBED), jnp.float32),
          pltpu.SemaphoreType.DMA,
          pltpu.SemaphoreType.DMA,
      ],
  )
  def k(idx_hbm, table_hbm, out_hbm, idx_v, buf0, buf1, sem0, sem1):
    wid = lax.axis_index("s") * _NC + lax.axis_index("c")
    g_base = wid * _NG_W
    row_base = wid * (_NG_W * _G)
    bufs = (buf0, buf1)
    sems = (sem0, sem1)

    pltpu.sync_copy(idx_hbm.at[pl.ds(g_base, _NG_W), :], idx_v)

    def start_gather(j, slot):
      pltpu.async_copy(table_hbm.at[idx_v.at[j]], bufs[slot], sems[slot])

    def wait_gather(j, slot):
      pltpu.make_async_copy(
          table_hbm.at[idx_v.at[j]], bufs[slot], sems[slot]).wait()

    def store(j, slot):
      pltpu.sync_copy(bufs[slot], out_hbm.at[pl.ds(row_base + j * _G, _G), :])

    start_gather(0, 0)

    @pl.loop(0, _NG_W - 2, step=2)
    def _(j0):
      for b in range(2):
        j = j0 + b
        start_gather(j + 1, (b + 1) % 2)
        wait_gather(j, b)
        store(j, b)

    start_gather(_NG_W - 1, 1)
    wait_gather(_NG_W - 2, 0)
    store(_NG_W - 2, 0)
    wait_gather(_NG_W - 1, 1)
    store(_NG_W - 1, 1)

  return k(idx2d, table)


def kernel(x, table):
  idx2d = jnp.transpose(x).reshape(_NG_TOTAL, _G).astype(jnp.int32)
  out = _sc_gather(idx2d, table)
  return out.reshape(SEQ, BATCH, EMBED)


# speedup vs baseline: 1.0212x; 1.0212x over previous
import functools

import jax
import jax.numpy as jnp
from jax import lax
from jax.experimental import pallas as pl
from jax.experimental.pallas import tpu as pltpu
from jax.experimental.pallas import tpu_sc as plsc

NUM_CLASSES = 1000000
EMBED = 64
BATCH = 4096
SEQ = 200

_NC = 2
_NS = 16
_NW = _NC * _NS
_ROWS = BATCH * SEQ
_G = 128
_NG_TOTAL = _ROWS // _G
_NG_W = _NG_TOTAL // _NW


_NBUF = 8
_A = 4


def _sc_gather(idx2d, table):
  mesh = plsc.VectorSubcoreMesh(core_axis_name="c", subcore_axis_name="s")

  @functools.partial(
      pl.kernel,
      out_type=jax.ShapeDtypeStruct((_ROWS, EMBED), jnp.float32),
      mesh=mesh,
      compiler_params=pltpu.CompilerParams(use_tc_tiling_on_sc=False),
      scratch_types=[
          pltpu.VMEM((_NG_W, _G), jnp.int32),
          [pltpu.VMEM((_G, EMBED), jnp.float32) for _ in range(_NBUF)],
          [pltpu.SemaphoreType.DMA for _ in range(_NBUF)],
          [pltpu.SemaphoreType.DMA for _ in range(_NBUF)],
      ],
  )
  def k(idx_hbm, table_hbm, out_hbm, idx_v, bufs, gsems, ssems):
    wid = lax.axis_index("s") * _NC + lax.axis_index("c")
    g_base = wid * _NG_W
    row_base = wid * (_NG_W * _G)

    pltpu.sync_copy(idx_hbm.at[pl.ds(g_base, _NG_W), :], idx_v)

    def start_gather(j, s):
      pltpu.async_copy(table_hbm.at[idx_v.at[j]], bufs[s], gsems[s])

    def wait_gather(j, s):
      pltpu.make_async_copy(
          table_hbm.at[idx_v.at[j]], bufs[s], gsems[s]).wait()

    def start_store(j, s):
      pltpu.async_copy(
          bufs[s], out_hbm.at[pl.ds(row_base + j * _G, _G), :], ssems[s])

    def wait_store(j, s):
      pltpu.make_async_copy(
          bufs[s], out_hbm.at[pl.ds(row_base + j * _G, _G), :], ssems[s]).wait()

    for j in range(_A):
      start_gather(j, j % _NBUF)

    for j in range(_NBUF - _A):
      wait_gather(j, j % _NBUF)
      start_store(j, j % _NBUF)
      start_gather(j + _A, (j + _A) % _NBUF)

    @pl.loop(_NBUF - _A, _NG_W - _A, step=_NBUF)
    def _(j0):
      for b in range(_NBUF):
        j = j0 + b
        s = (_NBUF - _A + b) % _NBUF
        sf = (s + _A) % _NBUF
        wait_gather(j, s)
        start_store(j, s)
        wait_store(j + _A - _NBUF, sf)
        start_gather(j + _A, sf)

    for j in range(_NG_W - _A, _NG_W):
      s = (_NBUF - _A + j - (_NG_W - _A)) % _NBUF
      wait_gather(j, s)
      start_store(j, s)
    for j in range(_NG_W - _NBUF, _NG_W):
      s = (_NBUF - _A + j - (_NG_W - _A)) % _NBUF
      wait_store(j, s)

  return k(idx2d, table)


def kernel(x, table):
  idx2d = jnp.transpose(x).reshape(_NG_TOTAL, _G).astype(jnp.int32)
  out = _sc_gather(idx2d, table)
  return out.reshape(SEQ, BATCH, EMBED)


# speedup vs baseline: 1.0244x; 1.0032x over previous
import functools

import jax
import jax.numpy as jnp
from jax import lax
from jax.experimental import pallas as pl
from jax.experimental.pallas import tpu as pltpu
from jax.experimental.pallas import tpu_sc as plsc

NUM_CLASSES = 1000000
EMBED = 64
BATCH = 4096
SEQ = 200

_NC = 2
_NS = 16
_NW = _NC * _NS
_G = BATCH // _NW
_NBUF = 8
_A = 4


def _sc_gather(idx3d, table):
  mesh = plsc.VectorSubcoreMesh(core_axis_name="c", subcore_axis_name="s")

  @functools.partial(
      pl.kernel,
      out_type=jax.ShapeDtypeStruct((SEQ, BATCH, EMBED), jnp.float32),
      mesh=mesh,
      compiler_params=pltpu.CompilerParams(use_tc_tiling_on_sc=False),
      scratch_types=[
          pltpu.VMEM((SEQ, _G), jnp.int32),
          [pltpu.VMEM((_G, EMBED), jnp.float32) for _ in range(_NBUF)],
          [pltpu.SemaphoreType.DMA for _ in range(_NBUF)],
          [pltpu.SemaphoreType.DMA for _ in range(_NBUF)],
      ],
  )
  def k(idx_hbm, table_hbm, out_hbm, idx_v, bufs, gsems, ssems):
    wid = lax.axis_index("s") * _NC + lax.axis_index("c")
    b_base = wid * _G

    pltpu.sync_copy(idx_hbm.at[:, wid, :], idx_v)

    def start_gather(l, s):
      pltpu.async_copy(table_hbm.at[idx_v.at[l]], bufs[s], gsems[s])

    def wait_gather(l, s):
      pltpu.make_async_copy(
          table_hbm.at[idx_v.at[l]], bufs[s], gsems[s]).wait()

    def start_store(l, s):
      pltpu.async_copy(
          bufs[s], out_hbm.at[l, pl.ds(b_base, _G), :], ssems[s])

    def wait_store(l, s):
      pltpu.make_async_copy(
          bufs[s], out_hbm.at[l, pl.ds(b_base, _G), :], ssems[s]).wait()

    for l in range(_A):
      start_gather(l, l % _NBUF)

    for l in range(_NBUF - _A):
      wait_gather(l, l % _NBUF)
      start_store(l, l % _NBUF)
      start_gather(l + _A, (l + _A) % _NBUF)

    @pl.loop(_NBUF - _A, SEQ - _A, step=_NBUF)
    def _(l0):
      for b in range(_NBUF):
        l = l0 + b
        s = (_NBUF - _A + b) % _NBUF
        sf = (s + _A) % _NBUF
        wait_gather(l, s)
        start_store(l, s)
        wait_store(l + _A - _NBUF, sf)
        start_gather(l + _A, sf)

    for l in range(SEQ - _A, SEQ):
      wait_gather(l, l % _NBUF)
      start_store(l, l % _NBUF)
    for l in range(SEQ - _NBUF, SEQ):
      wait_store(l, l % _NBUF)

  return k(idx3d, table)


def kernel(x, table):
  idx3d = jnp.transpose(x).reshape(SEQ, _NW, _G).astype(jnp.int32)
  out = _sc_gather(idx3d, table)
  return out
